# baseline (speedup 1.0000x reference)
import functools

import jax
import jax.numpy as jnp
from jax import lax
from jax.experimental import pallas as pl
from jax.experimental.pallas import tpu as pltpu
from jax.experimental.pallas import tpu_sc as plsc

BATCH = 16384
SEQ = 50
DIM = 64
NC = 2
NS = 16
NW = NC * NS
CB = 128
NCB = BATCH // CB
CPW = NCB // NW
NBUF = CPW


def _transpose_block(buf_g, buf_t):
    lane = jax.lax.iota(jnp.int32, 16)
    bases = [(lane + 16 * q) * CB for q in range(4)]
    for i in range(CB):
        for q in range(4):
            v = buf_g[i, pl.ds(16 * q, 16)]
            plsc.store_scatter(buf_t, [bases[q] + i], v)


def _emb_body(idx_hbm, w_hbm, out_hbm, idx_v, bufs_g, bufs_t, gsems, ssems,
              wid):
    base = wid * CPW * CB
    pltpu.sync_copy(idx_hbm.at[pl.ds(0, SEQ), pl.ds(base, CPW * CB)], idx_v)

    def gather(s, b):
        return pltpu.make_async_copy(
            w_hbm.at[idx_v.at[s, pl.ds(CB * b, CB)]], bufs_g[b], gsems[b])

    def stores(s, b):
        c = wid * CPW + b
        return [pltpu.make_async_copy(
            bufs_t[b].at[pl.ds(r * 8 * CB, 8 * CB)], out_hbm.at[s, r, c],
            ssems[b]) for r in range(8)]

    for b in range(NBUF):
        gather(0, b).start()

    def step(s, _):
        for b in range(NBUF):
            gather(s, b).wait()

            @pl.when(s > 0)
            def _():
                for st in stores(s - 1, b):
                    st.wait()

            _transpose_block(bufs_g[b], bufs_t[b])
            for st in stores(s, b):
                st.start()

            @pl.when(s < SEQ - 1)
            def _():
                gather(s + 1, b).start()

        return 0

    lax.fori_loop(0, SEQ, step, 0)

    for b in range(NBUF):
        for st in stores(SEQ - 1, b):
            st.wait()


def kernel(token_ids, weight):
    idx_t = token_ids.T

    mesh = plsc.VectorSubcoreMesh(core_axis_name="c", subcore_axis_name="s")

    @functools.partial(
        pl.kernel,
        mesh=mesh,
        out_type=jax.ShapeDtypeStruct((SEQ, 8, NCB, 8 * CB), jnp.float32),
        compiler_params=pltpu.CompilerParams(
            use_tc_tiling_on_sc=False, needs_layout_passes=False),
        scratch_types=[
            pltpu.VMEM((SEQ, CPW * CB), jnp.int32),
            *[pltpu.VMEM((CB, DIM), jnp.float32) for _ in range(NBUF)],
            *[pltpu.VMEM((DIM * CB,), jnp.float32) for _ in range(NBUF)],
            *[pltpu.SemaphoreType.DMA for _ in range(2 * NBUF)],
        ],
    )
    def emb(idx_hbm, w_hbm, out_hbm, idx_v, *rest):
        bufs_g = rest[:NBUF]
        bufs_t = rest[NBUF:2 * NBUF]
        gsems = rest[2 * NBUF:3 * NBUF]
        ssems = rest[3 * NBUF:]
        wid = lax.axis_index("s") * NC + lax.axis_index("c")
        _emb_body(idx_hbm, w_hbm, out_hbm, idx_v, bufs_g, bufs_t,
                  gsems, ssems, wid)

    out4 = emb(idx_t, weight)
    out5 = out4.reshape(SEQ, 8, NCB, 8, CB)
    return out5.transpose(2, 4, 0, 1, 3).reshape(BATCH, SEQ, DIM)


# speedup vs baseline: 1.3788x; 1.3788x over previous
import functools

import jax
import jax.numpy as jnp
from jax import lax
from jax.experimental import pallas as pl
from jax.experimental.pallas import tpu as pltpu
from jax.experimental.pallas import tpu_sc as plsc

BATCH = 16384
SEQ = 50
DIM = 64
NC = 2
NS = 16
NW = NC * NS
CB = 128
NCB = BATCH // CB
CPW = NCB // NW
NBUF = CPW


def _transpose_block(buf_g, buf_t):
    lane = jax.lax.iota(jnp.int32, 16)
    rows = [lane + 16 * q for q in range(4)]
    zero = lane * 0
    for i in range(CB):
        for q in range(4):
            v = buf_g[i, pl.ds(16 * q, 16)]
            plsc.store_scatter(buf_t, [rows[q], zero + i], v)


def _emb_body(idx_hbm, w_hbm, out_hbm, idx_v, bufs_g, bufs_t, gsems, ssems,
              wid):
    base = wid * CPW * CB
    pltpu.sync_copy(idx_hbm.at[pl.ds(0, SEQ), pl.ds(base, CPW * CB)], idx_v)

    def gather(s, b):
        return pltpu.make_async_copy(
            w_hbm.at[idx_v.at[s, pl.ds(CB * b, CB)]], bufs_g[b], gsems[b])

    def stores(s, b):
        c = wid * CPW + b
        return [pltpu.make_async_copy(
            bufs_t[b].at[pl.ds(8 * r, 8), pl.ds(0, CB)], out_hbm.at[s, r, c],
            ssems[b]) for r in range(8)]

    for b in range(NBUF):
        gather(0, b).start()

    def step(s, _):
        for b in range(NBUF):
            gather(s, b).wait()

            @pl.when(s > 0)
            def _():
                for st in stores(s - 1, b):
                    st.wait()

            _transpose_block(bufs_g[b], bufs_t[b])
            for st in stores(s, b):
                st.start()

            @pl.when(s < SEQ - 1)
            def _():
                gather(s + 1, b).start()

        return 0

    lax.fori_loop(0, SEQ, step, 0)

    for b in range(NBUF):
        for st in stores(SEQ - 1, b):
            st.wait()


def kernel(token_ids, weight):
    idx_t = token_ids.T

    mesh = plsc.VectorSubcoreMesh(core_axis_name="c", subcore_axis_name="s")

    @functools.partial(
        pl.kernel,
        mesh=mesh,
        out_type=jax.ShapeDtypeStruct((SEQ, 8, NCB, 8, CB), jnp.float32),
        compiler_params=pltpu.CompilerParams(
            use_tc_tiling_on_sc=False, needs_layout_passes=False),
        scratch_types=[
            pltpu.VMEM((SEQ, CPW * CB), jnp.int32),
            *[pltpu.VMEM((CB, DIM), jnp.float32) for _ in range(NBUF)],
            *[pltpu.VMEM((DIM, CB + 1), jnp.float32) for _ in range(NBUF)],
            *[pltpu.SemaphoreType.DMA for _ in range(2 * NBUF)],
        ],
    )
    def emb(idx_hbm, w_hbm, out_hbm, idx_v, *rest):
        bufs_g = rest[:NBUF]
        bufs_t = rest[NBUF:2 * NBUF]
        gsems = rest[2 * NBUF:3 * NBUF]
        ssems = rest[3 * NBUF:]
        wid = lax.axis_index("s") * NC + lax.axis_index("c")
        _emb_body(idx_hbm, w_hbm, out_hbm, idx_v, bufs_g, bufs_t,
                  gsems, ssems, wid)

    out5 = emb(idx_t, weight)
    return out5.transpose(2, 4, 0, 1, 3).reshape(BATCH, SEQ, DIM)
